# baseline (speedup 1.0000x reference)
import functools

import jax
import jax.numpy as jnp
from jax import lax
from jax.experimental import pallas as pl
from jax.experimental.pallas import tpu as pltpu
from jax.experimental.pallas import tpu_sc as plsc

_NC = 2
_NS = 16
_NW = _NC * _NS
_L = 16


@functools.lru_cache(maxsize=None)
def _make_gather(B: int, V: int, D: int, chunk: int, H: int):
    assert B % (_NW * chunk) == 0 and chunk % _L == 0 and chunk % H == 0
    bpw = B // _NW
    nchunk = bpw // chunk
    bchunk = chunk // H

    mesh = plsc.VectorSubcoreMesh(core_axis_name="c", subcore_axis_name="s")

    @functools.partial(
        pl.kernel,
        out_type=jax.ShapeDtypeStruct((B // H, H, D), jnp.float32),
        mesh=mesh,
        scratch_types=[
            pltpu.VMEM((chunk,), jnp.int32),
            pltpu.VMEM((chunk,), jnp.int32),
            pltpu.VMEM((chunk, D), jnp.float32),
            pltpu.VMEM((chunk, D), jnp.float32),
            pltpu.SemaphoreType.DMA,
            pltpu.SemaphoreType.DMA,
            pltpu.SemaphoreType.DMA,
            pltpu.SemaphoreType.DMA,
            pltpu.SemaphoreType.DMA,
            pltpu.SemaphoreType.DMA,
        ],
        compiler_params=pltpu.CompilerParams(use_tc_tiling_on_sc=False,
                                             needs_layout_passes=False),
    )
    def gather(idx_hbm, table_hbm, out_hbm, idx0, idx1, rows0, rows1,
               isem0, isem1, gsem0, gsem1, osem0, osem1):
        idx_v = (idx0, idx1)
        rows_v = (rows0, rows1)
        isem = (isem0, isem1)
        gsem = (gsem0, gsem1)
        osem = (osem0, osem1)
        wid = lax.axis_index("s") * _NC + lax.axis_index("c")
        base = wid * bpw

        def idx_start(ci, p):
            off = base + ci * chunk
            return pltpu.async_copy(idx_hbm.at[pl.ds(off, chunk)], idx_v[p],
                                    isem[p])

        def out_start(ci, p):
            b0 = (base + ci * chunk) // H
            return [
                pltpu.async_copy(rows_v[p].at[pl.ds(k * H, H)],
                                 out_hbm.at[b0 + k], osem[p])
                for k in range(bchunk)
            ]

        pend_idx = idx_start(0, 0)
        pend_gather = [None, None]
        pend_out = [None, None]

        for ci in range(nchunk):
            p = ci % 2
            pend_idx.wait()

            def add1(i, c):
                sl = pl.ds(i * _L, _L)
                idx_v[p][sl] = idx_v[p][sl] + 1
                return c

            lax.fori_loop(0, chunk // _L, add1, 0)
            if pend_out[p] is not None:
                for h in pend_out[p]:
                    h.wait()
                pend_out[p] = None
            pend_gather[p] = pltpu.async_copy(table_hbm.at[idx_v[p]],
                                              rows_v[p], gsem[p])
            if pend_gather[1 - p] is not None:
                pend_gather[1 - p].wait()
                pend_gather[1 - p] = None
                pend_out[1 - p] = out_start(ci - 1, 1 - p)
            if ci + 1 < nchunk:
                pend_idx = idx_start(ci + 1, 1 - p)

        pl_last = (nchunk - 1) % 2
        pend_gather[pl_last].wait()
        pend_out[pl_last] = out_start(nchunk - 1, pl_last)
        for p in range(2):
            if pend_out[p] is not None:
                for h in pend_out[p]:
                    h.wait()

    return gather


def kernel(inputs, table):
    b, h = inputs.shape
    v, d = table.shape
    idx = inputs.reshape(-1)
    return _make_gather(b * h, v, d, 1600, h)(idx, table)


# speedup vs baseline: 1.0019x; 1.0019x over previous
import functools

import jax
import jax.numpy as jnp
from jax import lax
from jax.experimental import pallas as pl
from jax.experimental.pallas import tpu as pltpu
from jax.experimental.pallas import tpu_sc as plsc

_NC = 2
_NS = 16
_NW = _NC * _NS
_L = 16


@functools.lru_cache(maxsize=None)
def _make_gather(B: int, V: int, D: int, chunk: int, H: int):
    assert B % (_NW * chunk) == 0 and chunk % _L == 0 and chunk % H == 0
    bpw = B // _NW
    nchunk = bpw // chunk
    bchunk = chunk // H

    mesh = plsc.VectorSubcoreMesh(core_axis_name="c", subcore_axis_name="s")

    @functools.partial(
        pl.kernel,
        out_type=jax.ShapeDtypeStruct((B // H, H, D), jnp.float32),
        mesh=mesh,
        scratch_types=[
            pltpu.VMEM((chunk,), jnp.int32),
            pltpu.VMEM((chunk,), jnp.int32),
            pltpu.VMEM((chunk, D), jnp.float32),
            pltpu.VMEM((chunk, D), jnp.float32),
            pltpu.SemaphoreType.DMA,
            pltpu.SemaphoreType.DMA,
            pltpu.SemaphoreType.DMA,
            pltpu.SemaphoreType.DMA,
            pltpu.SemaphoreType.DMA,
            pltpu.SemaphoreType.DMA,
        ],
        compiler_params=pltpu.CompilerParams(use_tc_tiling_on_sc=False),
    )
    def gather(idx_hbm, table_hbm, out_hbm, idx0, idx1, rows0, rows1,
               isem0, isem1, gsem0, gsem1, osem0, osem1):
        idx_v = (idx0, idx1)
        rows_v = (rows0, rows1)
        isem = (isem0, isem1)
        gsem = (gsem0, gsem1)
        osem = (osem0, osem1)
        wid = lax.axis_index("s") * _NC + lax.axis_index("c")
        base = wid * bpw

        def idx_start(ci, p):
            off = base + ci * chunk
            return pltpu.async_copy(idx_hbm.at[pl.ds(off, chunk)], idx_v[p],
                                    isem[p])

        def out_start(ci, p):
            b0 = (base + ci * chunk) // H
            return [
                pltpu.async_copy(rows_v[p].at[pl.ds(k * H, H)],
                                 out_hbm.at[b0 + k], osem[p])
                for k in range(bchunk)
            ]

        pend_idx = idx_start(0, 0)
        pend_gather = [None, None]
        pend_out = [None, None]

        for ci in range(nchunk):
            p = ci % 2
            pend_idx.wait()

            def add1(i, c):
                sl = pl.ds(i * _L, _L)
                idx_v[p][sl] = idx_v[p][sl] + 1
                return c

            lax.fori_loop(0, chunk // _L, add1, 0)
            if pend_out[p] is not None:
                for h in pend_out[p]:
                    h.wait()
                pend_out[p] = None
            pend_gather[p] = pltpu.async_copy(table_hbm.at[idx_v[p]],
                                              rows_v[p], gsem[p])
            if pend_gather[1 - p] is not None:
                pend_gather[1 - p].wait()
                pend_gather[1 - p] = None
                pend_out[1 - p] = out_start(ci - 1, 1 - p)
            if ci + 1 < nchunk:
                pend_idx = idx_start(ci + 1, 1 - p)

        pl_last = (nchunk - 1) % 2
        pend_gather[pl_last].wait()
        pend_out[pl_last] = out_start(nchunk - 1, pl_last)
        for p in range(2):
            if pend_out[p] is not None:
                for h in pend_out[p]:
                    h.wait()

    return gather


def kernel(inputs, table):
    b, h = inputs.shape
    v, d = table.shape
    idx = inputs.reshape(-1)
    return _make_gather(b * h, v, d, 1600, h)(idx, table)
